# baseline (speedup 1.0000x reference)
import functools

import jax
import jax.numpy as jnp
from jax import lax
from jax.experimental import pallas as pl
from jax.experimental.pallas import tpu as pltpu
from jax.experimental.pallas import tpu_sc as plsc

NU, NI, E, D, H, OUTD = 10000, 10000, 320000, 128, 128, 16
NA = NU + NI
NC, NS = 2, 16
EW = E // NS
CHUNK = 80
NCH = EW // CHUNK
RPT = 624
TAIL0 = NS * RPT


def _agg_body(*refs, gather):
    if gather:
        (h_hbm, src_hbm, dst_hbm, out_sum,
         src_a, dst_a, src_b, dst_b, rows_a, rows_b, acc_sh,
         sem_ga, sem_gb, sem_sa, sem_sb) = refs
    else:
        (dst_hbm, out_sum,
         dst_a, dst_b, rows_a, ones_v, acc_sh, sem_sa, sem_sb) = refs
    c = lax.axis_index("c")
    s = lax.axis_index("s")
    r0 = pl.multiple_of(s * RPT, 8)
    tb = pl.multiple_of(TAIL0 + (s % 2) * 8, 8)

    z16 = jnp.zeros((16,), jnp.float32)
    one16 = jnp.ones((16,), jnp.float32)

    def fill(i, carry):
        for j in range(H // 16):
            rows_a[i, pl.ds(j * 16, 16)] = z16
            if not gather:
                ones_v[i, pl.ds(j * 16, 16)] = one16
        return carry
    lax.fori_loop(0, CHUNK, fill, None)

    for t in range(7):
        pltpu.sync_copy(rows_a, acc_sh.at[pl.ds(r0 + t * CHUNK, CHUNK)])
    pltpu.sync_copy(rows_a.at[pl.ds(0, 64)], acc_sh.at[pl.ds(r0 + 560, 64)])
    pltpu.sync_copy(rows_a.at[pl.ds(0, 8)], acc_sh.at[pl.ds(tb, 8)])
    plsc.subcore_barrier()

    ebase = (c * NS + s) * EW

    def load_idx(k, sv, dv):
        base = pl.multiple_of(ebase + k * CHUNK, 8)
        if gather:
            pltpu.sync_copy(src_hbm.at[pl.ds(base, CHUNK)], sv)
        pltpu.sync_copy(dst_hbm.at[pl.ds(base, CHUNK)], dv)

    ga_start = lambda: pltpu.async_copy(h_hbm.at[src_a], rows_a, sem_ga)
    gb_start = lambda: pltpu.async_copy(h_hbm.at[src_b], rows_b, sem_gb)
    ga_wait = lambda: pltpu.make_async_copy(h_hbm.at[src_a], rows_a, sem_ga).wait()
    gb_wait = lambda: pltpu.make_async_copy(h_hbm.at[src_b], rows_b, sem_gb).wait()
    upd_a = rows_a if gather else ones_v
    upd_b = rows_b if gather else ones_v
    sa_start = lambda: pltpu.async_copy(upd_a, acc_sh.at[dst_a], sem_sa, add=True)
    sb_start = lambda: pltpu.async_copy(upd_b, acc_sh.at[dst_b], sem_sb, add=True)
    sa_wait = lambda: pltpu.make_async_copy(upd_a, acc_sh.at[dst_a], sem_sa).wait()
    sb_wait = lambda: pltpu.make_async_copy(upd_b, acc_sh.at[dst_b], sem_sb).wait()

    load_idx(0, src_a if gather else None, dst_a)
    if gather:
        ga_start()
        ga_wait()
    sa_start()
    load_idx(1, src_b if gather else None, dst_b)
    if gather:
        gb_start()
    sa_wait()
    load_idx(2, src_a if gather else None, dst_a)
    if gather:
        ga_start()
        gb_wait()
    sb_start()

    def pair(g, carry):
        k = 2 * g
        if gather:
            ga_wait()
        sa_start()
        sb_wait()
        load_idx(k + 1, src_b if gather else None, dst_b)
        if gather:
            gb_start()
        sa_wait()
        load_idx(k + 2, src_a if gather else None, dst_a)
        if gather:
            ga_start()
            gb_wait()
        sb_start()
        return carry
    lax.fori_loop(1, NCH // 2 - 1, pair, None)

    if gather:
        ga_wait()
    sa_start()
    sb_wait()
    load_idx(NCH - 1, src_b if gather else None, dst_b)
    if gather:
        gb_start()
    sa_wait()
    if gather:
        gb_wait()
    sb_start()
    sb_wait()
    plsc.subcore_barrier()

    for t in range(7):
        o = pl.multiple_of(r0 + t * CHUNK, 8)
        pltpu.sync_copy(acc_sh.at[pl.ds(o, CHUNK)], rows_a)
        pltpu.sync_copy(rows_a, out_sum.at[c, pl.ds(o, CHUNK)])
    o = pl.multiple_of(r0 + 560, 8)
    pltpu.sync_copy(acc_sh.at[pl.ds(o, 64)], rows_a.at[pl.ds(0, 64)])
    pltpu.sync_copy(rows_a.at[pl.ds(0, 64)], out_sum.at[c, pl.ds(o, 64)])
    pltpu.sync_copy(acc_sh.at[pl.ds(tb, 8)], rows_a.at[pl.ds(0, 8)])
    pltpu.sync_copy(rows_a.at[pl.ds(0, 8)], out_sum.at[c, pl.ds(tb, 8)])


def _make_agg(gather):
    mesh = plsc.VectorSubcoreMesh(core_axis_name="c", subcore_axis_name="s")
    if gather:
        scratch = [
            pltpu.VMEM((CHUNK,), jnp.int32),
            pltpu.VMEM((CHUNK,), jnp.int32),
            pltpu.VMEM((CHUNK,), jnp.int32),
            pltpu.VMEM((CHUNK,), jnp.int32),
            pltpu.VMEM((CHUNK, H), jnp.float32),
            pltpu.VMEM((CHUNK, H), jnp.float32),
            pltpu.VMEM_SHARED((NU, H), jnp.float32),
            pltpu.SemaphoreType.DMA,
            pltpu.SemaphoreType.DMA,
            pltpu.SemaphoreType.DMA,
            pltpu.SemaphoreType.DMA,
        ]
    else:
        scratch = [
            pltpu.VMEM((CHUNK,), jnp.int32),
            pltpu.VMEM((CHUNK,), jnp.int32),
            pltpu.VMEM((CHUNK, H), jnp.float32),
            pltpu.VMEM((CHUNK, H), jnp.float32),
            pltpu.VMEM_SHARED((NU, H), jnp.float32),
            pltpu.SemaphoreType.DMA,
            pltpu.SemaphoreType.DMA,
        ]
    return pl.kernel(
        functools.partial(_agg_body, gather=gather),
        mesh=mesh,
        out_type=jax.ShapeDtypeStruct((NC, NU, H), jnp.float32),
        scratch_types=scratch,
    )


_RB = 1000


def _enc_kernel(x_ref, w_ref, b_ref, o_ref):
    o_ref[...] = jnp.maximum(
        jnp.dot(x_ref[...], w_ref[0], preferred_element_type=jnp.float32)
        + b_ref[0], 0.0)


def _encode(x_all, w_st, b_st):
    return pl.pallas_call(
        _enc_kernel,
        grid=(NA // _RB,),
        in_specs=[
            pl.BlockSpec((_RB, D), lambda i: (i, 0)),
            pl.BlockSpec((1, D, H), lambda i: (i // (NU // _RB), 0, 0)),
            pl.BlockSpec((1, 1, H), lambda i: (i // (NU // _RB), 0, 0)),
        ],
        out_specs=pl.BlockSpec((_RB, H), lambda i: (i, 0)),
        out_shape=jax.ShapeDtypeStruct((NA, H), jnp.float32),
    )(x_all, w_st, b_st)


def _sage_kernel(sum_ref, cnt_ref, hall_ref, wl_ref, bl_ref, wr_ref, o_ref):
    cblk = cnt_ref[:, 0:1]
    mean = sum_ref[...] / jnp.maximum(cblk, 1.0)
    hall = hall_ref[...]
    out = (jnp.dot(mean, wl_ref[0], preferred_element_type=jnp.float32)
           + bl_ref[0]
           + jnp.dot(hall, wr_ref[0], preferred_element_type=jnp.float32))
    nrm = jnp.sqrt(jnp.sum(out * out, axis=-1, keepdims=True))
    out = out / jnp.maximum(nrm, 1e-12)
    o_ref[...] = jnp.maximum(out, 0.0) + hall


def _sage_finish(sums, cnts, h_all, wl_st, bl_st, wr_st):
    return pl.pallas_call(
        _sage_kernel,
        grid=(NA // _RB,),
        in_specs=[
            pl.BlockSpec((_RB, H), lambda i: (i, 0)),
            pl.BlockSpec((_RB, H), lambda i: (i, 0)),
            pl.BlockSpec((_RB, H), lambda i: (i, 0)),
            pl.BlockSpec((1, H, H), lambda i: (i // (NU // _RB), 0, 0)),
            pl.BlockSpec((1, 1, H), lambda i: (i // (NU // _RB), 0, 0)),
            pl.BlockSpec((1, H, H), lambda i: (i // (NU // _RB), 0, 0)),
        ],
        out_specs=pl.BlockSpec((_RB, H), lambda i: (i, 0)),
        out_shape=jax.ShapeDtypeStruct((NA, H), jnp.float32),
    )(sums, cnts, h_all, wl_st, bl_st, wr_st)


def _head_kernel(x_ref, w1_ref, b1_ref, w2_ref, b2_ref, o_ref):
    z = jnp.maximum(
        jnp.dot(x_ref[...], w1_ref[...], preferred_element_type=jnp.float32)
        + b1_ref[...], 0.0)
    o_ref[...] = (jnp.dot(z, w2_ref[...], preferred_element_type=jnp.float32)
                  + b2_ref[...])


def _head(h_all, w1, b1, w2, b2):
    hh = w1.shape[1]
    return pl.pallas_call(
        _head_kernel,
        grid=(NU // _RB,),
        in_specs=[
            pl.BlockSpec((_RB, H), lambda i: (i, 0)),
            pl.BlockSpec((H, hh), lambda i: (0, 0)),
            pl.BlockSpec((1, hh), lambda i: (0, 0)),
            pl.BlockSpec((hh, OUTD), lambda i: (0, 0)),
            pl.BlockSpec((1, OUTD), lambda i: (0, 0)),
        ],
        out_specs=pl.BlockSpec((_RB, OUTD), lambda i: (i, 0)),
        out_shape=jax.ShapeDtypeStruct((NU, OUTD), jnp.float32),
    )(h_all, w1, b1.reshape(1, hh), w2, b2.reshape(1, OUTD))


def kernel(x_user, x_item, edge_index_user_to_item, edge_index_item_to_user,
           enc_user_w, enc_user_b, enc_item_w, enc_item_b,
           u2i_wl0, u2i_bl0, u2i_wr0, i2u_wl0, i2u_bl0, i2u_wr0,
           u2i_wl1, u2i_bl1, u2i_wr1, i2u_wl1, i2u_bl1, i2u_wr1,
           head_w1, head_b1, head_w2, head_b2):
    agg = _make_agg(gather=True)
    deg = _make_agg(gather=False)

    src_st = jnp.concatenate(
        [edge_index_item_to_user[0] + NU, edge_index_user_to_item[0]])
    dst_st = jnp.concatenate(
        [edge_index_item_to_user[1], edge_index_user_to_item[1]])

    x_all = jnp.concatenate([x_user, x_item])
    enc_w = jnp.stack([enc_user_w, enc_item_w])
    enc_b = jnp.stack([enc_user_b.reshape(1, H), enc_item_b.reshape(1, H)])
    h_all = _encode(x_all, enc_w, enc_b)

    cnt = deg(dst_st).reshape(NA, H)

    layer_w = (
        (i2u_wl0, i2u_bl0, i2u_wr0, u2i_wl0, u2i_bl0, u2i_wr0),
        (i2u_wl1, i2u_bl1, i2u_wr1, u2i_wl1, u2i_bl1, u2i_wr1),
    )
    for (wl_u, bl_u, wr_u, wl_i, bl_i, wr_i) in layer_w:
        sums = agg(h_all, src_st, dst_st).reshape(NA, H)
        wl_st = jnp.stack([wl_u, wl_i])
        bl_st = jnp.stack([bl_u.reshape(1, H), bl_i.reshape(1, H)])
        wr_st = jnp.stack([wr_u, wr_i])
        h_all = _sage_finish(sums, cnt, h_all, wl_st, bl_st, wr_st)

    return _head(h_all, head_w1, head_b1, head_w2, head_b2)


# speedup vs baseline: 1.2781x; 1.2781x over previous
import functools

import jax
import jax.numpy as jnp
from jax import lax
from jax.experimental import pallas as pl
from jax.experimental.pallas import tpu as pltpu
from jax.experimental.pallas import tpu_sc as plsc

NU, NI, E, D, H, OUTD = 10000, 10000, 320000, 128, 128, 16
NA = NU + NI
NC, NS = 2, 16
EW = E // NS
CHUNK = 80
NCH = EW // CHUNK
RPT = 624
TAIL0 = NS * RPT


def _agg_body(*refs, gather):
    if gather:
        (h_hbm, src_hbm, dst_hbm, out_sum,
         src_all, dst_a, dst_b, rows_a, rows_b, acc_sh,
         sem_ga, sem_gb, sem_sa, sem_sb, sem_ia, sem_ib) = refs
    else:
        (dst_hbm, out_sum,
         dst_a, dst_b, rows_a, ones_v, acc_sh,
         sem_sa, sem_sb, sem_ia, sem_ib) = refs
    c = lax.axis_index("c")
    s = lax.axis_index("s")
    r0 = pl.multiple_of(s * RPT, 8)
    tb = pl.multiple_of(TAIL0 + (s % 2) * 8, 8)

    z16 = jnp.zeros((16,), jnp.float32)
    one16 = jnp.ones((16,), jnp.float32)

    def fill(i, carry):
        for j in range(H // 16):
            rows_a[i, pl.ds(j * 16, 16)] = z16
            if not gather:
                ones_v[i, pl.ds(j * 16, 16)] = one16
        return carry
    lax.fori_loop(0, CHUNK, fill, None)

    for t in range(7):
        pltpu.sync_copy(rows_a, acc_sh.at[pl.ds(r0 + t * CHUNK, CHUNK)])
    pltpu.sync_copy(rows_a.at[pl.ds(0, 64)], acc_sh.at[pl.ds(r0 + 560, 64)])
    pltpu.sync_copy(rows_a.at[pl.ds(0, 8)], acc_sh.at[pl.ds(tb, 8)])
    plsc.subcore_barrier()

    ebase = (c * NS + s) * EW

    if gather:
        pltpu.sync_copy(src_hbm.at[pl.ds(pl.multiple_of(ebase, 8), EW)],
                        src_all)

    def src_slice(k):
        return src_all.at[pl.ds(pl.multiple_of(k * CHUNK, 8), CHUNK)]

    def ia_start(k):
        base = pl.multiple_of(ebase + k * CHUNK, 8)
        pltpu.async_copy(dst_hbm.at[pl.ds(base, CHUNK)], dst_a, sem_ia)

    def ib_start(k):
        base = pl.multiple_of(ebase + k * CHUNK, 8)
        pltpu.async_copy(dst_hbm.at[pl.ds(base, CHUNK)], dst_b, sem_ib)

    ia_wait = lambda: pltpu.make_async_copy(dst_hbm.at[pl.ds(0, CHUNK)],
                                            dst_a, sem_ia).wait()
    ib_wait = lambda: pltpu.make_async_copy(dst_hbm.at[pl.ds(0, CHUNK)],
                                            dst_b, sem_ib).wait()
    ga_start = lambda k: pltpu.async_copy(h_hbm.at[src_slice(k)], rows_a, sem_ga)
    gb_start = lambda k: pltpu.async_copy(h_hbm.at[src_slice(k)], rows_b, sem_gb)
    ga_wait = lambda: pltpu.make_async_copy(h_hbm.at[src_slice(0)], rows_a,
                                            sem_ga).wait()
    gb_wait = lambda: pltpu.make_async_copy(h_hbm.at[src_slice(0)], rows_b,
                                            sem_gb).wait()
    upd_a = rows_a if gather else ones_v
    upd_b = rows_b if gather else ones_v
    sa_start = lambda: pltpu.async_copy(upd_a, acc_sh.at[dst_a], sem_sa, add=True)
    sb_start = lambda: pltpu.async_copy(upd_b, acc_sh.at[dst_b], sem_sb, add=True)
    sa_wait = lambda: pltpu.make_async_copy(upd_a, acc_sh.at[dst_a], sem_sa).wait()
    sb_wait = lambda: pltpu.make_async_copy(upd_b, acc_sh.at[dst_b], sem_sb).wait()

    ia_start(0)
    ib_start(1)
    if gather:
        ga_start(0)
        ga_wait()
    ia_wait()
    sa_start()
    if gather:
        gb_start(1)
    sa_wait()
    ia_start(2)
    if gather:
        ga_start(2)
        gb_wait()
    ib_wait()
    sb_start()

    def pair(g, carry):
        k = 2 * g
        if gather:
            ga_wait()
        ia_wait()
        sa_start()
        sb_wait()
        ib_start(k + 1)
        if gather:
            gb_start(k + 1)
        sa_wait()
        ia_start(k + 2)
        if gather:
            ga_start(k + 2)
            gb_wait()
        ib_wait()
        sb_start()
        return carry
    lax.fori_loop(1, NCH // 2 - 1, pair, None)

    if gather:
        ga_wait()
    ia_wait()
    sa_start()
    sb_wait()
    ib_start(NCH - 1)
    if gather:
        gb_start(NCH - 1)
    sa_wait()
    if gather:
        gb_wait()
    ib_wait()
    sb_start()
    sb_wait()
    plsc.subcore_barrier()

    for t in range(7):
        o = pl.multiple_of(r0 + t * CHUNK, 8)
        pltpu.sync_copy(acc_sh.at[pl.ds(o, CHUNK)], rows_a)
        pltpu.sync_copy(rows_a, out_sum.at[c, pl.ds(o, CHUNK)])
    o = pl.multiple_of(r0 + 560, 8)
    pltpu.sync_copy(acc_sh.at[pl.ds(o, 64)], rows_a.at[pl.ds(0, 64)])
    pltpu.sync_copy(rows_a.at[pl.ds(0, 64)], out_sum.at[c, pl.ds(o, 64)])
    pltpu.sync_copy(acc_sh.at[pl.ds(tb, 8)], rows_a.at[pl.ds(0, 8)])
    pltpu.sync_copy(rows_a.at[pl.ds(0, 8)], out_sum.at[c, pl.ds(tb, 8)])


def _make_agg(gather):
    mesh = plsc.VectorSubcoreMesh(core_axis_name="c", subcore_axis_name="s")
    if gather:
        scratch = [
            pltpu.VMEM((EW,), jnp.int32),
            pltpu.VMEM((CHUNK,), jnp.int32),
            pltpu.VMEM((CHUNK,), jnp.int32),
            pltpu.VMEM((CHUNK, H), jnp.float32),
            pltpu.VMEM((CHUNK, H), jnp.float32),
            pltpu.VMEM_SHARED((NU, H), jnp.float32),
            pltpu.SemaphoreType.DMA,
            pltpu.SemaphoreType.DMA,
            pltpu.SemaphoreType.DMA,
            pltpu.SemaphoreType.DMA,
            pltpu.SemaphoreType.DMA,
            pltpu.SemaphoreType.DMA,
        ]
    else:
        scratch = [
            pltpu.VMEM((CHUNK,), jnp.int32),
            pltpu.VMEM((CHUNK,), jnp.int32),
            pltpu.VMEM((CHUNK, H), jnp.float32),
            pltpu.VMEM((CHUNK, H), jnp.float32),
            pltpu.VMEM_SHARED((NU, H), jnp.float32),
            pltpu.SemaphoreType.DMA,
            pltpu.SemaphoreType.DMA,
            pltpu.SemaphoreType.DMA,
            pltpu.SemaphoreType.DMA,
        ]
    return pl.kernel(
        functools.partial(_agg_body, gather=gather),
        mesh=mesh,
        out_type=jax.ShapeDtypeStruct((NC, NU, H), jnp.float32),
        scratch_types=scratch,
    )


_RB = 1000


def _enc_kernel(x_ref, w_ref, b_ref, o_ref):
    o_ref[...] = jnp.maximum(
        jnp.dot(x_ref[...], w_ref[0], preferred_element_type=jnp.float32)
        + b_ref[0], 0.0)


def _encode(x_all, w_st, b_st):
    return pl.pallas_call(
        _enc_kernel,
        grid=(NA // _RB,),
        in_specs=[
            pl.BlockSpec((_RB, D), lambda i: (i, 0)),
            pl.BlockSpec((1, D, H), lambda i: (i // (NU // _RB), 0, 0)),
            pl.BlockSpec((1, 1, H), lambda i: (i // (NU // _RB), 0, 0)),
        ],
        out_specs=pl.BlockSpec((_RB, H), lambda i: (i, 0)),
        out_shape=jax.ShapeDtypeStruct((NA, H), jnp.float32),
    )(x_all, w_st, b_st)


def _sage_kernel(sum_ref, cnt_ref, hall_ref, wl_ref, bl_ref, wr_ref, o_ref):
    cblk = cnt_ref[:, 0:1]
    mean = sum_ref[...] / jnp.maximum(cblk, 1.0)
    hall = hall_ref[...]
    out = (jnp.dot(mean, wl_ref[0], preferred_element_type=jnp.float32)
           + bl_ref[0]
           + jnp.dot(hall, wr_ref[0], preferred_element_type=jnp.float32))
    nrm = jnp.sqrt(jnp.sum(out * out, axis=-1, keepdims=True))
    out = out / jnp.maximum(nrm, 1e-12)
    o_ref[...] = jnp.maximum(out, 0.0) + hall


def _sage_finish(sums, cnts, h_all, wl_st, bl_st, wr_st):
    return pl.pallas_call(
        _sage_kernel,
        grid=(NA // _RB,),
        in_specs=[
            pl.BlockSpec((_RB, H), lambda i: (i, 0)),
            pl.BlockSpec((_RB, H), lambda i: (i, 0)),
            pl.BlockSpec((_RB, H), lambda i: (i, 0)),
            pl.BlockSpec((1, H, H), lambda i: (i // (NU // _RB), 0, 0)),
            pl.BlockSpec((1, 1, H), lambda i: (i // (NU // _RB), 0, 0)),
            pl.BlockSpec((1, H, H), lambda i: (i // (NU // _RB), 0, 0)),
        ],
        out_specs=pl.BlockSpec((_RB, H), lambda i: (i, 0)),
        out_shape=jax.ShapeDtypeStruct((NA, H), jnp.float32),
    )(sums, cnts, h_all, wl_st, bl_st, wr_st)


def _head_kernel(x_ref, w1_ref, b1_ref, w2_ref, b2_ref, o_ref):
    z = jnp.maximum(
        jnp.dot(x_ref[...], w1_ref[...], preferred_element_type=jnp.float32)
        + b1_ref[...], 0.0)
    o_ref[...] = (jnp.dot(z, w2_ref[...], preferred_element_type=jnp.float32)
                  + b2_ref[...])


def _head(h_all, w1, b1, w2, b2):
    hh = w1.shape[1]
    return pl.pallas_call(
        _head_kernel,
        grid=(NU // _RB,),
        in_specs=[
            pl.BlockSpec((_RB, H), lambda i: (i, 0)),
            pl.BlockSpec((H, hh), lambda i: (0, 0)),
            pl.BlockSpec((1, hh), lambda i: (0, 0)),
            pl.BlockSpec((hh, OUTD), lambda i: (0, 0)),
            pl.BlockSpec((1, OUTD), lambda i: (0, 0)),
        ],
        out_specs=pl.BlockSpec((_RB, OUTD), lambda i: (i, 0)),
        out_shape=jax.ShapeDtypeStruct((NU, OUTD), jnp.float32),
    )(h_all, w1, b1.reshape(1, hh), w2, b2.reshape(1, OUTD))


def kernel(x_user, x_item, edge_index_user_to_item, edge_index_item_to_user,
           enc_user_w, enc_user_b, enc_item_w, enc_item_b,
           u2i_wl0, u2i_bl0, u2i_wr0, i2u_wl0, i2u_bl0, i2u_wr0,
           u2i_wl1, u2i_bl1, u2i_wr1, i2u_wl1, i2u_bl1, i2u_wr1,
           head_w1, head_b1, head_w2, head_b2):
    agg = _make_agg(gather=True)
    deg = _make_agg(gather=False)

    src_st = jnp.concatenate(
        [edge_index_item_to_user[0] + NU, edge_index_user_to_item[0]])
    dst_st = jnp.concatenate(
        [edge_index_item_to_user[1], edge_index_user_to_item[1]])

    x_all = jnp.concatenate([x_user, x_item])
    enc_w = jnp.stack([enc_user_w, enc_item_w])
    enc_b = jnp.stack([enc_user_b.reshape(1, H), enc_item_b.reshape(1, H)])
    h_all = _encode(x_all, enc_w, enc_b)

    cnt = deg(dst_st).reshape(NA, H)

    layer_w = (
        (i2u_wl0, i2u_bl0, i2u_wr0, u2i_wl0, u2i_bl0, u2i_wr0),
        (i2u_wl1, i2u_bl1, i2u_wr1, u2i_wl1, u2i_bl1, u2i_wr1),
    )
    for (wl_u, bl_u, wr_u, wl_i, bl_i, wr_i) in layer_w:
        sums = agg(h_all, src_st, dst_st).reshape(NA, H)
        wl_st = jnp.stack([wl_u, wl_i])
        bl_st = jnp.stack([bl_u.reshape(1, H), bl_i.reshape(1, H)])
        wr_st = jnp.stack([wr_u, wr_i])
        h_all = _sage_finish(sums, cnt, h_all, wl_st, bl_st, wr_st)

    return _head(h_all, head_w1, head_b1, head_w2, head_b2)


# speedup vs baseline: 1.3804x; 1.0800x over previous
import functools

import jax
import jax.numpy as jnp
from jax import lax
from jax.experimental import pallas as pl
from jax.experimental.pallas import tpu as pltpu
from jax.experimental.pallas import tpu_sc as plsc

NU, NI, E, D, H, OUTD = 10000, 10000, 320000, 128, 128, 16
NA = NU + NI
NC, NS = 2, 16
EW = E // NS
CHUNK = 128
NCH = 156
TAILE = EW - NCH * CHUNK
RPT = 624
TAIL0 = NS * RPT


def _agg_body(*refs, gather):
    if gather:
        (h_hbm, src_hbm, dst_hbm, out_sum,
         src_a, src_b, dst_a, dst_b, src_t, dst_t, rows_a, rows_b, acc_sh,
         sem_ga, sem_gb, sem_sa, sem_sb, sem_ia, sem_ib) = refs
    else:
        (dst_hbm, out_sum,
         dst_a, dst_b, dst_t, rows_a, ones_v, acc_sh,
         sem_sa, sem_sb, sem_ia, sem_ib) = refs
    c = lax.axis_index("c")
    s = lax.axis_index("s")
    r0 = pl.multiple_of(s * RPT, 8)
    tb = pl.multiple_of(TAIL0 + (s % 2) * 8, 8)

    z16 = jnp.zeros((16,), jnp.float32)
    one16 = jnp.ones((16,), jnp.float32)

    def fill(i, carry):
        for j in range(H // 16):
            rows_a[i, pl.ds(j * 16, 16)] = z16
            if not gather:
                ones_v[i, pl.ds(j * 16, 16)] = one16
        return carry
    lax.fori_loop(0, CHUNK, fill, None)

    for t in range(4):
        pltpu.sync_copy(rows_a, acc_sh.at[pl.ds(r0 + t * CHUNK, CHUNK)])
    pltpu.sync_copy(rows_a.at[pl.ds(0, 112)], acc_sh.at[pl.ds(r0 + 512, 112)])
    pltpu.sync_copy(rows_a.at[pl.ds(0, 8)], acc_sh.at[pl.ds(tb, 8)])
    plsc.subcore_barrier()

    ebase = (c * NS + s) * EW

    def ia_start(k):
        base = pl.multiple_of(ebase + k * CHUNK, 8)
        if gather:
            pltpu.async_copy(src_hbm.at[pl.ds(base, CHUNK)], src_a, sem_ia)
        pltpu.async_copy(dst_hbm.at[pl.ds(base, CHUNK)], dst_a, sem_ia)

    def ib_start(k):
        base = pl.multiple_of(ebase + k * CHUNK, 8)
        if gather:
            pltpu.async_copy(src_hbm.at[pl.ds(base, CHUNK)], src_b, sem_ib)
        pltpu.async_copy(dst_hbm.at[pl.ds(base, CHUNK)], dst_b, sem_ib)

    def ia_wait():
        if gather:
            pltpu.make_async_copy(src_hbm.at[pl.ds(0, CHUNK)], src_a,
                                  sem_ia).wait()
        pltpu.make_async_copy(dst_hbm.at[pl.ds(0, CHUNK)], dst_a,
                              sem_ia).wait()

    def ib_wait():
        if gather:
            pltpu.make_async_copy(src_hbm.at[pl.ds(0, CHUNK)], src_b,
                                  sem_ib).wait()
        pltpu.make_async_copy(dst_hbm.at[pl.ds(0, CHUNK)], dst_b,
                              sem_ib).wait()

    ga_start = lambda: pltpu.async_copy(h_hbm.at[src_a], rows_a, sem_ga)
    gb_start = lambda: pltpu.async_copy(h_hbm.at[src_b], rows_b, sem_gb)
    ga_wait = lambda: pltpu.make_async_copy(h_hbm.at[src_a], rows_a,
                                            sem_ga).wait()
    gb_wait = lambda: pltpu.make_async_copy(h_hbm.at[src_b], rows_b,
                                            sem_gb).wait()
    upd_a = rows_a if gather else ones_v
    upd_b = rows_b if gather else ones_v
    sa_start = lambda: pltpu.async_copy(upd_a, acc_sh.at[dst_a], sem_sa,
                                        add=True)
    sb_start = lambda: pltpu.async_copy(upd_b, acc_sh.at[dst_b], sem_sb,
                                        add=True)
    sa_wait = lambda: pltpu.make_async_copy(upd_a, acc_sh.at[dst_a],
                                            sem_sa).wait()
    sb_wait = lambda: pltpu.make_async_copy(upd_b, acc_sh.at[dst_b],
                                            sem_sb).wait()

    ia_start(0)
    ib_start(1)
    ia_wait()
    if gather:
        ga_start()
        ga_wait()
    sa_start()
    ib_wait()
    if gather:
        gb_start()
    sa_wait()
    ia_start(2)
    if gather:
        gb_wait()
    sb_start()

    def pair(g, carry):
        k = 2 * g
        ia_wait()
        if gather:
            ga_start()
            ga_wait()
        sb_wait()
        sa_start()
        ib_start(k + 1)
        ib_wait()
        if gather:
            gb_start()
        sa_wait()
        ia_start(k + 2)
        if gather:
            gb_wait()
        sb_start()
        return carry
    lax.fori_loop(1, NCH // 2 - 1, pair, None)

    ia_wait()
    if gather:
        ga_start()
        ga_wait()
    sb_wait()
    sa_start()
    ib_start(NCH - 1)
    ib_wait()
    if gather:
        gb_start()
    sa_wait()
    tbase = pl.multiple_of(ebase + NCH * CHUNK, 8)
    if gather:
        pltpu.sync_copy(src_hbm.at[pl.ds(tbase, TAILE)], src_t)
    pltpu.sync_copy(dst_hbm.at[pl.ds(tbase, TAILE)], dst_t)
    if gather:
        pltpu.sync_copy(h_hbm.at[src_t], rows_a.at[pl.ds(0, TAILE)])
        gb_wait()
    sb_start()
    upd_t = rows_a.at[pl.ds(0, TAILE)] if gather else ones_v.at[pl.ds(0, TAILE)]
    pltpu.sync_copy(upd_t, acc_sh.at[dst_t], add=True)
    sb_wait()
    plsc.subcore_barrier()

    for t in range(4):
        o = pl.multiple_of(r0 + t * CHUNK, 8)
        pltpu.sync_copy(acc_sh.at[pl.ds(o, CHUNK)], rows_a)
        pltpu.sync_copy(rows_a, out_sum.at[c, pl.ds(o, CHUNK)])
    o = pl.multiple_of(r0 + 512, 8)
    pltpu.sync_copy(acc_sh.at[pl.ds(o, 112)], rows_a.at[pl.ds(0, 112)])
    pltpu.sync_copy(rows_a.at[pl.ds(0, 112)], out_sum.at[c, pl.ds(o, 112)])
    pltpu.sync_copy(acc_sh.at[pl.ds(tb, 8)], rows_a.at[pl.ds(0, 8)])
    pltpu.sync_copy(rows_a.at[pl.ds(0, 8)], out_sum.at[c, pl.ds(tb, 8)])


def _make_agg(gather):
    mesh = plsc.VectorSubcoreMesh(core_axis_name="c", subcore_axis_name="s")
    if gather:
        scratch = [
            pltpu.VMEM((CHUNK,), jnp.int32),
            pltpu.VMEM((CHUNK,), jnp.int32),
            pltpu.VMEM((CHUNK,), jnp.int32),
            pltpu.VMEM((CHUNK,), jnp.int32),
            pltpu.VMEM((TAILE,), jnp.int32),
            pltpu.VMEM((TAILE,), jnp.int32),
            pltpu.VMEM((CHUNK, H), jnp.float32),
            pltpu.VMEM((CHUNK, H), jnp.float32),
            pltpu.VMEM_SHARED((NU, H), jnp.float32),
            pltpu.SemaphoreType.DMA,
            pltpu.SemaphoreType.DMA,
            pltpu.SemaphoreType.DMA,
            pltpu.SemaphoreType.DMA,
            pltpu.SemaphoreType.DMA,
            pltpu.SemaphoreType.DMA,
        ]
    else:
        scratch = [
            pltpu.VMEM((CHUNK,), jnp.int32),
            pltpu.VMEM((CHUNK,), jnp.int32),
            pltpu.VMEM((TAILE,), jnp.int32),
            pltpu.VMEM((CHUNK, H), jnp.float32),
            pltpu.VMEM((CHUNK, H), jnp.float32),
            pltpu.VMEM_SHARED((NU, H), jnp.float32),
            pltpu.SemaphoreType.DMA,
            pltpu.SemaphoreType.DMA,
            pltpu.SemaphoreType.DMA,
            pltpu.SemaphoreType.DMA,
        ]
    return pl.kernel(
        functools.partial(_agg_body, gather=gather),
        mesh=mesh,
        out_type=jax.ShapeDtypeStruct((NC, NU, H), jnp.float32),
        scratch_types=scratch,
    )


_RB = 1000
_NBU = NU // _RB


def _enc_kernel(x_ref, w_ref, b_ref, o_ref):
    o_ref[...] = jnp.maximum(
        jnp.dot(x_ref[...], w_ref[0], preferred_element_type=jnp.float32)
        + b_ref[0], 0.0)


def _encode(x_all, w_st, b_st):
    return pl.pallas_call(
        _enc_kernel,
        grid=(NA // _RB,),
        in_specs=[
            pl.BlockSpec((_RB, D), lambda i: (i, 0)),
            pl.BlockSpec((1, D, H), lambda i: (i // _NBU, 0, 0)),
            pl.BlockSpec((1, 1, H), lambda i: (i // _NBU, 0, 0)),
        ],
        out_specs=pl.BlockSpec((_RB, H), lambda i: (i, 0)),
        out_shape=jax.ShapeDtypeStruct((NA, H), jnp.float32),
    )(x_all, w_st, b_st)


def _sage_common(sum_ref, cnt_ref, hall_ref, wl_ref, bl_ref, wr_ref):
    cblk = cnt_ref[0, :, 0:1]
    mean = sum_ref[0] / jnp.maximum(cblk, 1.0)
    hall = hall_ref[...]
    out = (jnp.dot(mean, wl_ref[0], preferred_element_type=jnp.float32)
           + bl_ref[0]
           + jnp.dot(hall, wr_ref[0], preferred_element_type=jnp.float32))
    nrm = jnp.sqrt(jnp.sum(out * out, axis=-1, keepdims=True))
    out = out / jnp.maximum(nrm, 1e-12)
    return jnp.maximum(out, 0.0) + hall


def _sage_kernel(sum_ref, cnt_ref, hall_ref, wl_ref, bl_ref, wr_ref, o_ref):
    o_ref[...] = _sage_common(sum_ref, cnt_ref, hall_ref, wl_ref, bl_ref,
                              wr_ref)


def _sage_finish(sums, cnts, h_all, wl_st, bl_st, wr_st):
    return pl.pallas_call(
        _sage_kernel,
        grid=(NA // _RB,),
        in_specs=[
            pl.BlockSpec((1, _RB, H), lambda i: (i // _NBU, i % _NBU, 0)),
            pl.BlockSpec((1, _RB, H), lambda i: (i // _NBU, i % _NBU, 0)),
            pl.BlockSpec((_RB, H), lambda i: (i, 0)),
            pl.BlockSpec((1, H, H), lambda i: (i // _NBU, 0, 0)),
            pl.BlockSpec((1, 1, H), lambda i: (i // _NBU, 0, 0)),
            pl.BlockSpec((1, H, H), lambda i: (i // _NBU, 0, 0)),
        ],
        out_specs=pl.BlockSpec((_RB, H), lambda i: (i, 0)),
        out_shape=jax.ShapeDtypeStruct((NA, H), jnp.float32),
    )(sums, cnts, h_all, wl_st, bl_st, wr_st)


def _sage_head_kernel(sum_ref, cnt_ref, hall_ref, wl_ref, bl_ref, wr_ref,
                      w1_ref, b1_ref, w2_ref, b2_ref, o_ref):
    h_new = _sage_common(sum_ref, cnt_ref, hall_ref, wl_ref, bl_ref, wr_ref)
    z = jnp.maximum(
        jnp.dot(h_new, w1_ref[...], preferred_element_type=jnp.float32)
        + b1_ref[...], 0.0)
    o_ref[...] = (jnp.dot(z, w2_ref[...], preferred_element_type=jnp.float32)
                  + b2_ref[...])


def _sage_head(sums, cnts, h_all, wl_st, bl_st, wr_st, w1, b1, w2, b2):
    hh = w1.shape[1]
    return pl.pallas_call(
        _sage_head_kernel,
        grid=(NA // _RB,),
        in_specs=[
            pl.BlockSpec((1, _RB, H), lambda i: (i // _NBU, i % _NBU, 0)),
            pl.BlockSpec((1, _RB, H), lambda i: (i // _NBU, i % _NBU, 0)),
            pl.BlockSpec((_RB, H), lambda i: (i, 0)),
            pl.BlockSpec((1, H, H), lambda i: (i // _NBU, 0, 0)),
            pl.BlockSpec((1, 1, H), lambda i: (i // _NBU, 0, 0)),
            pl.BlockSpec((1, H, H), lambda i: (i // _NBU, 0, 0)),
            pl.BlockSpec((H, hh), lambda i: (0, 0)),
            pl.BlockSpec((1, hh), lambda i: (0, 0)),
            pl.BlockSpec((hh, OUTD), lambda i: (0, 0)),
            pl.BlockSpec((1, OUTD), lambda i: (0, 0)),
        ],
        out_specs=pl.BlockSpec((_RB, OUTD), lambda i: (i, 0)),
        out_shape=jax.ShapeDtypeStruct((NA, OUTD), jnp.float32),
    )(sums, cnts, h_all, wl_st, bl_st, wr_st,
      w1, b1.reshape(1, hh), w2, b2.reshape(1, OUTD))[:NU]


def kernel(x_user, x_item, edge_index_user_to_item, edge_index_item_to_user,
           enc_user_w, enc_user_b, enc_item_w, enc_item_b,
           u2i_wl0, u2i_bl0, u2i_wr0, i2u_wl0, i2u_bl0, i2u_wr0,
           u2i_wl1, u2i_bl1, u2i_wr1, i2u_wl1, i2u_bl1, i2u_wr1,
           head_w1, head_b1, head_w2, head_b2):
    agg = _make_agg(gather=True)
    deg = _make_agg(gather=False)

    src_st = jnp.concatenate(
        [edge_index_item_to_user[0] + NU, edge_index_user_to_item[0]])
    dst_st = jnp.concatenate(
        [edge_index_item_to_user[1], edge_index_user_to_item[1]])

    x_all = jnp.concatenate([x_user, x_item])
    enc_w = jnp.stack([enc_user_w, enc_item_w])
    enc_b = jnp.stack([enc_user_b.reshape(1, H), enc_item_b.reshape(1, H)])
    h_all = _encode(x_all, enc_w, enc_b)

    cnt = deg(dst_st)

    wl0 = jnp.stack([i2u_wl0, u2i_wl0])
    bl0 = jnp.stack([i2u_bl0.reshape(1, H), u2i_bl0.reshape(1, H)])
    wr0 = jnp.stack([i2u_wr0, u2i_wr0])
    wl1 = jnp.stack([i2u_wl1, u2i_wl1])
    bl1 = jnp.stack([i2u_bl1.reshape(1, H), u2i_bl1.reshape(1, H)])
    wr1 = jnp.stack([i2u_wr1, u2i_wr1])

    sums0 = agg(h_all, src_st, dst_st)
    h_all = _sage_finish(sums0, cnt, h_all, wl0, bl0, wr0)
    sums1 = agg(h_all, src_st, dst_st)
    return _sage_head(sums1, cnt, h_all, wl1, bl1, wr1,
                      head_w1, head_b1, head_w2, head_b2)
